# baseline (speedup 1.0000x reference)
import jax
import jax.numpy as jnp
from jax import lax
from jax.experimental import pallas as pl
from jax.experimental.pallas import tpu as pltpu
from jax.experimental.pallas import tpu_sc as plsc

NUM_SPIKES = 128
LANES = 16
ROWS = 2
NBUF = 2


def _spike_body(x_hbm, r_hbm, out_hbm, xvs, rvs, ovs, sin, sout):
    B, C, H, W = x_hbm.shape
    S = NUM_SPIKES
    n_workers = 32
    wpb = n_workers // B
    rows_pw = H // wpb
    n_chunks = rows_pw // ROWS
    n_full = ROWS * (W // LANES)

    wid = lax.axis_index("s") * 2 + lax.axis_index("c")
    b = wid // wpb
    row_base = (wid % wpb) * rows_pw

    iot = lax.iota(jnp.int32, LANES)
    rem_i = jnp.where(iot >= 8, 1, 0).astype(jnp.int32)
    rem_j = (W // LANES) * LANES + (iot & 7)

    def splat(v):
        return jnp.full((LANES,), v, jnp.int32)

    def in_slices(j):
        rsl = pl.ds(row_base + j * ROWS, ROWS)
        return x_hbm.at[b, :, rsl, :], r_hbm.at[b, :, rsl, :]

    def issue_in(j, t):
        xs, rs = in_slices(j)
        pltpu.async_copy(xs, xvs[t], sin[t])
        pltpu.async_copy(rs, rvs[t], sin[t])

    def wait_in(j, t):
        xs, rs = in_slices(j)
        pltpu.make_async_copy(xs, xvs[t], sin[t]).wait()
        pltpu.make_async_copy(rs, rvs[t], sin[t]).wait()

    def out_slice(j):
        return out_hbm.at[b, :, pl.ds(row_base + j * ROWS, ROWS), :]

    issue_in(0, 0)

    def chunk_pair(j2, _):
        for t in range(NBUF):
            j = j2 * NBUF + t
            xv, rv, ov = xvs[t], rvs[t], ovs[t]

            @pl.when(j < n_chunks)
            def _():
                @pl.when(j + 1 < n_chunks)
                def _():
                    issue_in(j + 1, (t + 1) % NBUF)

                wait_in(j, t)

                @pl.when(j >= NBUF)
                def _():
                    pltpu.make_async_copy(ov, out_slice(j - NBUF), sout[t]).wait()

                for g in range(n_full):
                    i = g // (W // LANES)
                    cb = (g % (W // LANES)) * LANES
                    sl = pl.ds(cb, LANES)
                    gi, gj = splat(i), cb + iot

                    def cum_body(c, acc):
                        acc = acc + xv[c, i, sl]
                        xv[c, i, sl] = acc
                        return acc

                    total = plsc.parallel_loop(
                        0, C, carry=jnp.zeros((LANES,), jnp.float32), unroll=8
                    )(cum_body)

                    @plsc.parallel_loop(0, S, unroll=4)
                    def spike_body(s):
                        v = rv[s, i, sl] * total
                        pos = jnp.zeros((LANES,), jnp.int32)
                        for k in (64, 32, 16, 8, 4, 2, 1):
                            probe = plsc.load_gather(xv, [pos + (k - 1), gi, gj])
                            pos = pos + jnp.where(probe < v, k, 0).astype(jnp.int32)
                        ov[s, i, sl] = pos

                def cum_body_r(c, acc):
                    acc = acc + plsc.load_gather(xv, [splat(c), rem_i, rem_j])
                    plsc.store_scatter(xv, [splat(c), rem_i, rem_j], acc)
                    return acc

                total = plsc.parallel_loop(
                    0, C, carry=jnp.zeros((LANES,), jnp.float32), unroll=8
                )(cum_body_r)

                @plsc.parallel_loop(0, S, unroll=4)
                def spike_body_r(s):
                    v = plsc.load_gather(rv, [splat(s), rem_i, rem_j]) * total
                    pos = jnp.zeros((LANES,), jnp.int32)
                    for k in (64, 32, 16, 8, 4, 2, 1):
                        probe = plsc.load_gather(xv, [pos + (k - 1), rem_i, rem_j])
                        pos = pos + jnp.where(probe < v, k, 0).astype(jnp.int32)
                    plsc.store_scatter(ov, [splat(s), rem_i, rem_j], pos)

                pltpu.async_copy(ov, out_slice(j), sout[t])
        return 0

    lax.fori_loop(0, (n_chunks + NBUF - 1) // NBUF, chunk_pair, 0)

    for t in range(NBUF):
        j = n_chunks - NBUF + t
        buf = j % NBUF
        pltpu.make_async_copy(ovs[buf], out_slice(j), sout[buf]).wait()


def kernel(input):
    B, C, H, W = input.shape
    rkey = jax.random.key(42)
    rand = jax.random.uniform(rkey, (B, NUM_SPIKES, H, W), dtype=input.dtype)

    mesh = plsc.VectorSubcoreMesh(
        core_axis_name="c", subcore_axis_name="s", num_cores=2, num_subcores=16
    )
    run = pl.kernel(
        _spike_body,
        out_type=jax.ShapeDtypeStruct((B, NUM_SPIKES, H, W), jnp.int32),
        mesh=mesh,
        scratch_types=[
            [pltpu.VMEM((C, ROWS, W), jnp.float32) for _ in range(NBUF)],
            [pltpu.VMEM((NUM_SPIKES, ROWS, W), jnp.float32) for _ in range(NBUF)],
            [pltpu.VMEM((NUM_SPIKES, ROWS, W), jnp.int32) for _ in range(NBUF)],
            [pltpu.SemaphoreType.DMA for _ in range(NBUF)],
            [pltpu.SemaphoreType.DMA for _ in range(NBUF)],
        ],
        compiler_params=pltpu.CompilerParams(use_tc_tiling_on_sc=False,
                                             needs_layout_passes=False),
    )
    return run(input, rand)


# speedup vs baseline: 1.5091x; 1.5091x over previous
import jax
import jax.numpy as jnp
from jax import lax
from jax.experimental import pallas as pl
from jax.experimental.pallas import tpu as pltpu
from jax.experimental.pallas import tpu_sc as plsc

NUM_SPIKES = 128
LANES = 16
ROWS = 2
NBUF = 2


def _spike_body(x_hbm, r_hbm, out_hbm, xvs, rvs, ovs, sin, sout):
    B, C, H, W = x_hbm.shape
    S = NUM_SPIKES
    CHUNK = ROWS * W
    n_workers = 32
    wpb = n_workers // B
    rows_pw = H // wpb
    n_chunks = rows_pw // ROWS
    n_full = ROWS * (W // LANES)

    wid = lax.axis_index("s") * 2 + lax.axis_index("c")
    b = wid // wpb
    row_base = (wid % wpb) * rows_pw
    pix_base = row_base * W

    iot = lax.iota(jnp.int32, LANES)
    rem_i = jnp.where(iot >= 8, 1, 0).astype(jnp.int32)
    rem_j = (W // LANES) * LANES + (iot & 7)
    rem_flat = rem_i * W + rem_j

    def splat(v):
        return jnp.full((LANES,), v, jnp.int32)

    def in_slices(j):
        xs = x_hbm.at[b, :, pl.ds(row_base + j * ROWS, ROWS), :]
        rs = r_hbm.at[b, :, pl.ds(pix_base + j * CHUNK, CHUNK)]
        return xs, rs

    def issue_in(j, t):
        xs, rs = in_slices(j)
        pltpu.async_copy(xs, xvs[t], sin[t])
        pltpu.async_copy(rs, rvs[t], sin[t])

    def wait_in(j, t):
        xs, rs = in_slices(j)
        pltpu.make_async_copy(xs, xvs[t], sin[t]).wait()
        pltpu.make_async_copy(rs, rvs[t], sin[t]).wait()

    def out_slice(j):
        return out_hbm.at[b, :, pl.ds(pix_base + j * CHUNK, CHUNK)]

    issue_in(0, 0)

    def chunk_pair(j2, _):
        for t in range(NBUF):
            j = j2 * NBUF + t
            xv, rv, ov = xvs[t], rvs[t], ovs[t]

            @pl.when(j < n_chunks)
            def _():
                @pl.when(j + 1 < n_chunks)
                def _():
                    issue_in(j + 1, (t + 1) % NBUF)

                wait_in(j, t)

                @pl.when(j >= NBUF)
                def _():
                    pltpu.make_async_copy(ov, out_slice(j - NBUF), sout[t]).wait()

                for g in range(n_full):
                    i = g // (W // LANES)
                    cb = (g % (W // LANES)) * LANES
                    sl = pl.ds(cb, LANES)
                    fsl = pl.ds(i * W + cb, LANES)
                    gi, gj = splat(i), cb + iot

                    def cum_body(c, acc):
                        acc = acc + xv[c, i, sl]
                        xv[c, i, sl] = acc
                        return acc

                    total = plsc.parallel_loop(
                        0, C, carry=jnp.zeros((LANES,), jnp.float32), unroll=8
                    )(cum_body)

                    @plsc.parallel_loop(0, S, unroll=4)
                    def spike_body(s):
                        v = rv[s, fsl] * total
                        pos = jnp.zeros((LANES,), jnp.int32)
                        for k in (64, 32, 16, 8, 4, 2, 1):
                            probe = plsc.load_gather(xv, [pos + (k - 1), gi, gj])
                            pos = pos + jnp.where(probe < v, k, 0).astype(jnp.int32)
                        ov[s, fsl] = pos

                def cum_body_r(c, acc):
                    acc = acc + plsc.load_gather(xv, [splat(c), rem_i, rem_j])
                    plsc.store_scatter(xv, [splat(c), rem_i, rem_j], acc)
                    return acc

                total = plsc.parallel_loop(
                    0, C, carry=jnp.zeros((LANES,), jnp.float32), unroll=8
                )(cum_body_r)

                @plsc.parallel_loop(0, S, unroll=4)
                def spike_body_r(s):
                    v = plsc.load_gather(rv, [splat(s), rem_flat]) * total
                    pos = jnp.zeros((LANES,), jnp.int32)
                    for k in (64, 32, 16, 8, 4, 2, 1):
                        probe = plsc.load_gather(xv, [pos + (k - 1), rem_i, rem_j])
                        pos = pos + jnp.where(probe < v, k, 0).astype(jnp.int32)
                    plsc.store_scatter(ov, [splat(s), rem_flat], pos)

                pltpu.async_copy(ov, out_slice(j), sout[t])
        return 0

    lax.fori_loop(0, (n_chunks + NBUF - 1) // NBUF, chunk_pair, 0)

    for t in range(NBUF):
        j = n_chunks - NBUF + t
        buf = j % NBUF
        pltpu.make_async_copy(ovs[buf], out_slice(j), sout[buf]).wait()


def kernel(input):
    B, C, H, W = input.shape
    P = H * W
    rkey = jax.random.key(42)
    rand = jax.random.uniform(rkey, (B, NUM_SPIKES, P), dtype=input.dtype)

    mesh = plsc.VectorSubcoreMesh(
        core_axis_name="c", subcore_axis_name="s", num_cores=2, num_subcores=16
    )
    run = pl.kernel(
        _spike_body,
        out_type=jax.ShapeDtypeStruct((B, NUM_SPIKES, P), jnp.int32),
        mesh=mesh,
        scratch_types=[
            [pltpu.VMEM((C, ROWS, W), jnp.float32) for _ in range(NBUF)],
            [pltpu.VMEM((NUM_SPIKES, ROWS * W), jnp.float32) for _ in range(NBUF)],
            [pltpu.VMEM((NUM_SPIKES, ROWS * W), jnp.int32) for _ in range(NBUF)],
            [pltpu.SemaphoreType.DMA for _ in range(NBUF)],
            [pltpu.SemaphoreType.DMA for _ in range(NBUF)],
        ],
        compiler_params=pltpu.CompilerParams(use_tc_tiling_on_sc=False,
                                             needs_layout_passes=False),
    )
    out = run(input, rand)
    return out.reshape(B, NUM_SPIKES, H, W)


# speedup vs baseline: 1.5242x; 1.0100x over previous
import jax
import jax.numpy as jnp
from jax import lax
from jax.experimental import pallas as pl
from jax.experimental.pallas import tpu as pltpu
from jax.experimental.pallas import tpu_sc as plsc

NUM_SPIKES = 128
LANES = 16
CHUNK = 112
NBUF = 2


def _spike_body(x_hbm, r_hbm, out_hbm, xvs, rvs, ovs, sin, sout):
    B, C, P = x_hbm.shape
    n_workers = 32
    wpb = n_workers // B
    cols_pw = P // wpb
    n_chunks = cols_pw // CHUNK
    groups = CHUNK // LANES

    wid = lax.axis_index("s") * 2 + lax.axis_index("c")
    b = wid // wpb
    base = (wid % wpb) * cols_pw

    def in_slices(j):
        sl = pl.ds(base + j * CHUNK, CHUNK)
        return x_hbm.at[b, :, sl], r_hbm.at[b, :, sl]

    def issue_in(j, t):
        xs, rs = in_slices(j)
        pltpu.async_copy(xs, xvs[t], sin[t])
        pltpu.async_copy(rs, rvs[t], sin[t])

    def wait_in(j, t):
        xs, rs = in_slices(j)
        pltpu.make_async_copy(xs, xvs[t], sin[t]).wait()
        pltpu.make_async_copy(rs, rvs[t], sin[t]).wait()

    def out_slice(j):
        return out_hbm.at[b, :, pl.ds(base + j * CHUNK, CHUNK)]

    issue_in(0, 0)

    def chunk_pair(j2, _):
        for t in range(NBUF):
            j = j2 * NBUF + t
            xv, rv, ov = xvs[t], rvs[t], ovs[t]

            @pl.when(j < n_chunks)
            def _():
                @pl.when(j + 1 < n_chunks)
                def _():
                    issue_in(j + 1, (t + 1) % NBUF)

                wait_in(j, t)

                @pl.when(j >= NBUF)
                def _():
                    pltpu.make_async_copy(ov, out_slice(j - NBUF), sout[t]).wait()

                for g in range(groups):
                    sl = pl.ds(g * LANES, LANES)
                    lanecol = lax.iota(jnp.int32, LANES) + (g * LANES)

                    def cum_body(c, acc):
                        acc = acc + xv[c, sl]
                        xv[c, sl] = acc
                        return acc

                    total = plsc.parallel_loop(
                        0, C, carry=jnp.zeros((LANES,), jnp.float32), unroll=8
                    )(cum_body)

                    @plsc.parallel_loop(0, NUM_SPIKES, unroll=8)
                    def spike_body(s):
                        v = rv[s, sl] * total
                        pos = jnp.zeros((LANES,), jnp.int32)
                        for k in (64, 32, 16, 8, 4, 2, 1):
                            probe = plsc.load_gather(xv, [pos + (k - 1), lanecol])
                            pos = pos + jnp.where(probe < v, k, 0).astype(jnp.int32)
                        ov[s, sl] = pos

                pltpu.async_copy(ov, out_slice(j), sout[t])
        return 0

    lax.fori_loop(0, (n_chunks + NBUF - 1) // NBUF, chunk_pair, 0)

    for t in range(NBUF):
        j = n_chunks - NBUF + t
        buf = j % NBUF
        pltpu.make_async_copy(ovs[buf], out_slice(j), sout[buf]).wait()


def kernel(input):
    B, C, H, W = input.shape
    P = H * W
    x = input.reshape(B, C, P)
    rkey = jax.random.key(42)
    rand = jax.random.uniform(rkey, (B, NUM_SPIKES, P), dtype=input.dtype)

    mesh = plsc.VectorSubcoreMesh(
        core_axis_name="c", subcore_axis_name="s", num_cores=2, num_subcores=16
    )
    run = pl.kernel(
        _spike_body,
        out_type=jax.ShapeDtypeStruct((B, NUM_SPIKES, P), jnp.int32),
        mesh=mesh,
        scratch_types=[
            [pltpu.VMEM((C, CHUNK), jnp.float32) for _ in range(NBUF)],
            [pltpu.VMEM((NUM_SPIKES, CHUNK), jnp.float32) for _ in range(NBUF)],
            [pltpu.VMEM((NUM_SPIKES, CHUNK), jnp.int32) for _ in range(NBUF)],
            [pltpu.SemaphoreType.DMA for _ in range(NBUF)],
            [pltpu.SemaphoreType.DMA for _ in range(NBUF)],
        ],
        compiler_params=pltpu.CompilerParams(use_tc_tiling_on_sc=False,
                                             needs_layout_passes=False),
    )
    out = run(x, rand)
    return out.reshape(B, NUM_SPIKES, H, W)


# speedup vs baseline: 1.6365x; 1.0737x over previous
import jax
import jax.numpy as jnp
from jax import lax
from jax.experimental import pallas as pl
from jax.experimental.pallas import tpu as pltpu
from jax.experimental.pallas import tpu_sc as plsc

NUM_SPIKES = 128
LANES = 16
CHUNK = 112
NBUF = 2


def _spike_body(x_hbm, r_hbm, out_hbm, xvs, rvs, ovs, sin, sout):
    B, C, P = x_hbm.shape
    n_workers = 32
    wpb = n_workers // B
    cols_pw = P // wpb
    n_chunks = cols_pw // CHUNK
    groups = CHUNK // LANES

    wid = lax.axis_index("s") * 2 + lax.axis_index("c")
    b = wid // wpb
    base = (wid % wpb) * cols_pw

    def in_slices(j):
        sl = pl.ds(base + j * CHUNK, CHUNK)
        return x_hbm.at[b, :, sl], r_hbm.at[b, :, sl]

    def issue_in(j, t):
        xs, rs = in_slices(j)
        pltpu.async_copy(xs, xvs[t], sin[t])
        pltpu.async_copy(rs, rvs[t], sin[t])

    def wait_in(j, t):
        xs, rs = in_slices(j)
        pltpu.make_async_copy(xs, xvs[t], sin[t]).wait()
        pltpu.make_async_copy(rs, rvs[t], sin[t]).wait()

    def out_slice(j):
        return out_hbm.at[b, :, pl.ds(base + j * CHUNK, CHUNK)]

    issue_in(0, 0)

    def chunk_pair(j2, _):
        for t in range(NBUF):
            j = j2 * NBUF + t
            xv, rv, ov = xvs[t], rvs[t], ovs[t]

            @pl.when(j < n_chunks)
            def _():
                @pl.when(j + 1 < n_chunks)
                def _():
                    issue_in(j + 1, (t + 1) % NBUF)

                wait_in(j, t)

                @pl.when(j >= NBUF)
                def _():
                    pltpu.make_async_copy(ov, out_slice(j - NBUF), sout[t]).wait()

                for g in range(groups):
                    sl = pl.ds(g * LANES, LANES)
                    lanecol = lax.iota(jnp.int32, LANES) + (g * LANES)

                    def cum_body(c, acc):
                        acc = acc + xv[c, sl]
                        xv[c, sl] = acc
                        return acc

                    total = plsc.parallel_loop(
                        0, C, carry=jnp.zeros((LANES,), jnp.float32), unroll=8
                    )(cum_body)

                    @plsc.parallel_loop(0, NUM_SPIKES, unroll=4)
                    def spike_body(s):
                        v = rv[s, sl] * total
                        pos = jnp.zeros((LANES,), jnp.int32)
                        for k in (64, 32, 16, 8, 4, 2, 1):
                            probe = plsc.load_gather(xv, [pos + (k - 1), lanecol])
                            pos = pos + jnp.where(probe < v, k, 0).astype(jnp.int32)
                        ov[s, sl] = pos

                pltpu.async_copy(ov, out_slice(j), sout[t])
        return 0

    lax.fori_loop(0, (n_chunks + NBUF - 1) // NBUF, chunk_pair, 0)

    for t in range(NBUF):
        j = n_chunks - NBUF + t
        buf = j % NBUF
        pltpu.make_async_copy(ovs[buf], out_slice(j), sout[buf]).wait()


def kernel(input):
    B, C, H, W = input.shape
    P = H * W
    x = input.reshape(B, C, P)
    rkey = jax.random.key(42)
    rand = jax.random.uniform(rkey, (B, NUM_SPIKES, P), dtype=input.dtype)

    mesh = plsc.VectorSubcoreMesh(
        core_axis_name="c", subcore_axis_name="s", num_cores=2, num_subcores=16
    )
    run = pl.kernel(
        _spike_body,
        out_type=jax.ShapeDtypeStruct((B, NUM_SPIKES, P), jnp.int32),
        mesh=mesh,
        scratch_types=[
            [pltpu.VMEM((C, CHUNK), jnp.float32) for _ in range(NBUF)],
            [pltpu.VMEM((NUM_SPIKES, CHUNK), jnp.float32) for _ in range(NBUF)],
            [pltpu.VMEM((NUM_SPIKES, CHUNK), jnp.int32) for _ in range(NBUF)],
            [pltpu.SemaphoreType.DMA for _ in range(NBUF)],
            [pltpu.SemaphoreType.DMA for _ in range(NBUF)],
        ],
        compiler_params=pltpu.CompilerParams(use_tc_tiling_on_sc=False,
                                             needs_layout_passes=False),
    )
    out = run(x, rand)
    return out.reshape(B, NUM_SPIKES, H, W)
